# baseline (speedup 1.0000x reference)
import jax
import jax.numpy as jnp
from jax import lax
from jax.experimental import pallas as pl
from jax.experimental.pallas import tpu as pltpu
from jax.experimental.pallas import tpu_sc as plsc


_SEQ_BLOCK = 512
_SEQ_SPLIT = 2560
_N_WORKERS = 32
_TAB_PAD = 64


def _pack_table(table):
    tb = table.astype(jnp.bfloat16)
    half = table.shape[1] // 2
    lo = lax.bitcast_convert_type(tb[:, :half], jnp.uint16).astype(jnp.uint32)
    hi = lax.bitcast_convert_type(tb[:, half:], jnp.uint16).astype(jnp.uint32)
    return lax.bitcast_convert_type((hi << 16) | lo, jnp.int32)


def _sc_gather_rows(table, idx):
    n_idx = idx.shape[0]
    w = table.shape[1]
    per_w = n_idx // _N_WORKERS
    mesh = plsc.VectorSubcoreMesh(core_axis_name="core", subcore_axis_name="subcore")

    @pl.kernel(
        out_type=jax.ShapeDtypeStruct((n_idx, w), table.dtype),
        mesh=mesh,
        scratch_types=[
            pltpu.VMEM((per_w,), jnp.int32),
            pltpu.VMEM((per_w, w), table.dtype),
            pltpu.SemaphoreType.DMA,
        ],
    )
    def gather_kernel(tab_hbm, i_hbm, o_hbm, idx_v, rows_v, sem):
        wid = jax.lax.axis_index("core") * 16 + jax.lax.axis_index("subcore")
        base = wid * per_w
        pltpu.sync_copy(i_hbm.at[pl.ds(base, per_w)], idx_v)
        pltpu.async_copy(tab_hbm.at[idx_v], rows_v, sem).wait()
        pltpu.sync_copy(rows_v, o_hbm.at[pl.ds(base, per_w)])

    return gather_kernel(table, idx)


def _tc_head_onehot_add(x, onehot, table_pad, d3):
    batch, seq, d = x.shape
    n_planes = onehot.shape[0]
    bs = _SEQ_BLOCK

    def body(x_ref, oh_ref, tab_ref, o_ref):
        tab = tab_ref[...]
        for c in range(n_planes):
            pe_c = jnp.dot(oh_ref[c], tab, preferred_element_type=jnp.float32)
            sl = slice(c * d3, (c + 1) * d3)
            for b in range(batch):
                o_ref[b, :, sl] = x_ref[b, :, sl] + pe_c

    return pl.pallas_call(
        body,
        grid=(_SEQ_SPLIT // bs,),
        in_specs=[
            pl.BlockSpec((batch, bs, d), lambda s: (0, s, 0)),
            pl.BlockSpec((n_planes, bs, _TAB_PAD), lambda s: (0, s, 0)),
            pl.BlockSpec((_TAB_PAD, d3), lambda s: (0, 0)),
        ],
        out_specs=pl.BlockSpec((batch, bs, d), lambda s: (0, s, 0)),
        out_shape=jax.ShapeDtypeStruct(x.shape, x.dtype),
    )(x, onehot, table_pad)


def _tc_tail_unpack_add(out_head, x, pe_packed, d3):
    batch, seq, d = x.shape
    n_planes, _, half = pe_packed.shape
    bs = _SEQ_BLOCK
    s0 = _SEQ_SPLIT // bs

    def body(prev_ref, x_ref, pe_ref, o_ref):
        del prev_ref
        for c in range(n_planes):
            word = pe_ref[c]
            pe_lo = lax.bitcast_convert_type(word << 16, jnp.float32)
            pe_hi = lax.bitcast_convert_type(word & (-65536), jnp.float32)
            sl_lo = slice(c * d3, c * d3 + half)
            sl_hi = slice(c * d3 + half, (c + 1) * d3)
            for b in range(batch):
                o_ref[b, :, sl_lo] = x_ref[b, :, sl_lo] + pe_lo
                o_ref[b, :, sl_hi] = x_ref[b, :, sl_hi] + pe_hi

    return pl.pallas_call(
        body,
        grid=((seq - _SEQ_SPLIT) // bs,),
        in_specs=[
            pl.BlockSpec(memory_space=pl.ANY),
            pl.BlockSpec((batch, bs, d), lambda s: (0, s + s0, 0)),
            pl.BlockSpec((n_planes, bs, half), lambda s: (0, s, 0)),
        ],
        out_specs=pl.BlockSpec((batch, bs, d), lambda s: (0, s + s0, 0)),
        out_shape=jax.ShapeDtypeStruct(x.shape, x.dtype),
        input_output_aliases={0: 0},
    )(out_head, x, pe_packed)


def kernel(x, src_tgt, Wx, Wy, Wz, src_pos_x, src_pos_y, src_pos_z,
           tgt_pos_x, tgt_pos_y, tgt_pos_z):
    batch, seq, d = x.shape
    d3 = Wx.shape[1]
    n_tab = Wx.shape[0] + Wy.shape[0] + Wz.shape[0]

    table = jnp.concatenate([Wx, Wy, Wz], axis=0)
    off_y = Wx.shape[0]
    off_z = off_y + Wy.shape[0]
    idx_src = jnp.concatenate([src_pos_x, src_pos_y + off_y, src_pos_z + off_z])
    idx_tgt = jnp.concatenate(
        [tgt_pos_x[:seq], tgt_pos_y[:seq] + off_y, tgt_pos_z[:seq] + off_z])
    idx = jnp.where(src_tgt, idx_src, idx_tgt).astype(jnp.int32).reshape(3, seq)

    table_pad = jnp.pad(table, ((0, _TAB_PAD - n_tab), (0, 0)))
    onehot = (idx[:, :_SEQ_SPLIT, None]
              == jax.lax.broadcasted_iota(jnp.int32, (1, 1, _TAB_PAD), 2)
              ).astype(jnp.float32)

    idx_tail = idx[:, _SEQ_SPLIT:].reshape(3 * (seq - _SEQ_SPLIT))
    pe_packed = _sc_gather_rows(_pack_table(table), idx_tail)
    pe_packed = pe_packed.reshape(3, seq - _SEQ_SPLIT, d3 // 2)

    out_head = _tc_head_onehot_add(x, onehot, table_pad, d3)
    return _tc_tail_unpack_add(out_head, x, pe_packed, d3)


# speedup vs baseline: 1.0467x; 1.0467x over previous
import jax
import jax.numpy as jnp
from jax import lax
from jax.experimental import pallas as pl
from jax.experimental.pallas import tpu as pltpu
from jax.experimental.pallas import tpu_sc as plsc


_SEQ_BLOCK = 512
_SEQ_SPLIT = 2048
_N_WORKERS = 32
_TAB_PAD = 64


def _pack_table(table):
    tb = table.astype(jnp.bfloat16)
    half = table.shape[1] // 2
    lo = lax.bitcast_convert_type(tb[:, :half], jnp.uint16).astype(jnp.uint32)
    hi = lax.bitcast_convert_type(tb[:, half:], jnp.uint16).astype(jnp.uint32)
    return lax.bitcast_convert_type((hi << 16) | lo, jnp.int32)


def _sc_gather_rows(table, idx):
    n_idx = idx.shape[0]
    w = table.shape[1]
    per_w = n_idx // _N_WORKERS
    mesh = plsc.VectorSubcoreMesh(core_axis_name="core", subcore_axis_name="subcore")

    @pl.kernel(
        out_type=jax.ShapeDtypeStruct((n_idx, w), table.dtype),
        mesh=mesh,
        scratch_types=[
            pltpu.VMEM((per_w,), jnp.int32),
            pltpu.VMEM((per_w, w), table.dtype),
            pltpu.SemaphoreType.DMA,
        ],
    )
    def gather_kernel(tab_hbm, i_hbm, o_hbm, idx_v, rows_v, sem):
        wid = jax.lax.axis_index("core") * 16 + jax.lax.axis_index("subcore")
        base = wid * per_w
        pltpu.sync_copy(i_hbm.at[pl.ds(base, per_w)], idx_v)
        pltpu.async_copy(tab_hbm.at[idx_v], rows_v, sem).wait()
        pltpu.sync_copy(rows_v, o_hbm.at[pl.ds(base, per_w)])

    return gather_kernel(table, idx)


def _tc_head_onehot_add(x, idx3d, table_pad, d3):
    batch, seq, d = x.shape
    n_planes = idx3d.shape[0]
    bs = _SEQ_BLOCK

    def body(x_ref, idx_ref, tab_ref, o_ref):
        tab = tab_ref[...]
        iot = lax.broadcasted_iota(jnp.int32, (bs, _TAB_PAD), 1)
        for c in range(n_planes):
            idx_c = idx_ref[c, 0, :].reshape(bs, 1)
            oh_c = (idx_c == iot).astype(jnp.float32)
            pe_c = jnp.dot(oh_c, tab, preferred_element_type=jnp.float32)
            sl = slice(c * d3, (c + 1) * d3)
            for b in range(batch):
                o_ref[b, :, sl] = x_ref[b, :, sl] + pe_c

    return pl.pallas_call(
        body,
        grid=(_SEQ_SPLIT // bs,),
        in_specs=[
            pl.BlockSpec((batch, bs, d), lambda s: (0, s, 0)),
            pl.BlockSpec((n_planes, 1, bs), lambda s: (0, 0, s)),
            pl.BlockSpec((_TAB_PAD, d3), lambda s: (0, 0)),
        ],
        out_specs=pl.BlockSpec((batch, bs, d), lambda s: (0, s, 0)),
        out_shape=jax.ShapeDtypeStruct(x.shape, x.dtype),
    )(x, idx3d, table_pad)


def _tc_tail_unpack_add(out_head, x, pe_packed, d3):
    batch, seq, d = x.shape
    n_planes, _, half = pe_packed.shape
    bs = _SEQ_BLOCK
    s0 = _SEQ_SPLIT // bs

    def body(prev_ref, x_ref, pe_ref, o_ref):
        del prev_ref
        for c in range(n_planes):
            word = pe_ref[c]
            pe_lo = lax.bitcast_convert_type(word << 16, jnp.float32)
            pe_hi = lax.bitcast_convert_type(word & (-65536), jnp.float32)
            sl_lo = slice(c * d3, c * d3 + half)
            sl_hi = slice(c * d3 + half, (c + 1) * d3)
            for b in range(batch):
                o_ref[b, :, sl_lo] = x_ref[b, :, sl_lo] + pe_lo
                o_ref[b, :, sl_hi] = x_ref[b, :, sl_hi] + pe_hi

    return pl.pallas_call(
        body,
        grid=((seq - _SEQ_SPLIT) // bs,),
        in_specs=[
            pl.BlockSpec(memory_space=pl.ANY),
            pl.BlockSpec((batch, bs, d), lambda s: (0, s + s0, 0)),
            pl.BlockSpec((n_planes, bs, half), lambda s: (0, s, 0)),
        ],
        out_specs=pl.BlockSpec((batch, bs, d), lambda s: (0, s + s0, 0)),
        out_shape=jax.ShapeDtypeStruct(x.shape, x.dtype),
        input_output_aliases={0: 0},
    )(out_head, x, pe_packed)


def kernel(x, src_tgt, Wx, Wy, Wz, src_pos_x, src_pos_y, src_pos_z,
           tgt_pos_x, tgt_pos_y, tgt_pos_z):
    batch, seq, d = x.shape
    d3 = Wx.shape[1]
    n_tab = Wx.shape[0] + Wy.shape[0] + Wz.shape[0]

    table = jnp.concatenate([Wx, Wy, Wz], axis=0)
    off_y = Wx.shape[0]
    off_z = off_y + Wy.shape[0]
    idx_src = jnp.concatenate([src_pos_x, src_pos_y + off_y, src_pos_z + off_z])
    idx_tgt = jnp.concatenate(
        [tgt_pos_x[:seq], tgt_pos_y[:seq] + off_y, tgt_pos_z[:seq] + off_z])
    idx = jnp.where(src_tgt, idx_src, idx_tgt).astype(jnp.int32).reshape(3, seq)

    table_pad = jnp.pad(table, ((0, _TAB_PAD - n_tab), (0, 0)))
    idx3d = idx[:, :_SEQ_SPLIT].reshape(3, 1, _SEQ_SPLIT)

    idx_tail = idx[:, _SEQ_SPLIT:].reshape(3 * (seq - _SEQ_SPLIT))
    pe_packed = _sc_gather_rows(_pack_table(table), idx_tail)
    pe_packed = pe_packed.reshape(3, seq - _SEQ_SPLIT, d3 // 2)

    out_head = _tc_head_onehot_add(x, idx3d, table_pad, d3)
    return _tc_tail_unpack_add(out_head, x, pe_packed, d3)
